# baseline (speedup 1.0000x reference)
import jax
import jax.numpy as jnp
from jax import lax
from jax.experimental import pallas as pl
from jax.experimental.pallas import tpu as pltpu
from jax.experimental.pallas import tpu_sc as plsc

VOCAB = 100000
D = 128
PAD_IDX = 0
CLS_IDX = 57255
SEP_IDX = 57256
EPS = 1e-12
B, L = 1024, 200
LP = 224
HU = 112
NW = 32
ROWS_PER_W = B // NW
NJ = D // 16
NCHUNK = LP // 16
UCHUNK = HU // 16
NPOS = 203
GROWS = (2 * VOCAB + D - 1) // D + 1


def _rsqrt16(x):
    i = lax.bitcast_convert_type(x, jnp.int32)
    i = jnp.int32(0x5F3759DF) - lax.shift_right_logical(i, 1)
    y = lax.bitcast_convert_type(i, jnp.float32)
    for _ in range(2):
        y = y * (1.5 - 0.5 * x * y * y)
    return y


def _sc_embed_body(ids_hbm, geo_hbm, tok_hbm, posf_hbm, par_hbm, out_hbm,
                   posc, par_v, idsA, idsB, gidxA, gidxB, pbA, pbB,
                   mfA, mfB, coA, coB, tokH0, tokH1, growH0, growH1,
                   outH0, outH1, semA, semB, isem, osem0, osem1):
    def allsum(v):
        iota = jnp.arange(16, dtype=jnp.int32)
        for k in (8, 4, 2, 1):
            v = v + jnp.take(v, jnp.bitwise_xor(iota, jnp.int32(k)), axis=0)
        return v

    wid = lax.axis_index("s") * 2 + lax.axis_index("c")
    row0 = wid * ROWS_PER_W

    pltpu.sync_copy(posf_hbm.at[pl.ds(0, 201 * D)], posc.at[pl.ds(0, 201 * D)])
    pltpu.sync_copy(posf_hbm.at[pl.ds(511 * D, D)], posc.at[pl.ds(201 * D, D)])
    pltpu.sync_copy(posf_hbm.at[pl.ds(512 * D, D)], posc.at[pl.ds(202 * D, D)])
    pltpu.sync_copy(par_hbm, par_v)

    z16 = jnp.zeros((16,), jnp.int32)
    for ids_v in (idsA, idsB):
        ids_v[pl.ds(200, 16)] = z16
        ids_v[pl.ds(208, 16)] = z16

    W0 = [par_v[pl.ds(0 * D + 16 * j, 16)] for j in range(NJ)]
    W1 = [par_v[pl.ds(1 * D + 16 * j, 16)] for j in range(NJ)]
    bg = [par_v[pl.ds(2 * D + 16 * j, 16)] for j in range(NJ)]
    gam = [par_v[pl.ds(3 * D + 16 * j, 16)] for j in range(NJ)]
    bet = [par_v[pl.ds(4 * D + 16 * j, 16)] for j in range(NJ)]

    def pid_row(ids_v, gidx_v, pb_v, mf_v, co_v):
        carry = jnp.float32(0)
        for j in range(NCHUNK):
            v = ids_v[pl.ds(j * 16, 16)]
            gidx_v[pl.ds(j * 16, 16)] = lax.shift_right_logical(v, 6)
            padf = jnp.where(v == PAD_IDX, jnp.float32(1), jnp.float32(0))
            clsf = jnp.where(v == CLS_IDX, jnp.float32(1), jnp.float32(0))
            sepf = jnp.where(v == SEP_IDX, jnp.float32(1), jnp.float32(0))
            mi_f = 1.0 - (padf + clsf + sepf)
            iota = jnp.arange(16, dtype=jnp.int32)
            cs = mi_f
            for k in (1, 2, 4, 8):
                sh = jnp.take(cs, jnp.maximum(iota - k, 0), axis=0)
                cs = cs + jnp.where(iota >= k, sh, jnp.float32(0))
            cs = cs + carry
            pidf = (cs * mi_f + jnp.float32(NPOS - 2) * clsf
                    + jnp.float32(NPOS - 1) * sepf)
            pb_v[pl.ds(j * 16, 16)] = (pidf * D).astype(jnp.int32)
            mf_v[pl.ds(j * 16, 16)] = mi_f
            coff = lax.shift_left(jnp.bitwise_and(v, jnp.int32(63)), 1)
            colb = jnp.bitwise_and(coff, jnp.int32(112))
            co_v[pl.ds(j * 16, 16)] = colb * 256 + (coff - colb)
            carry = cs[15]

    def issue_unit(ids_v, gidx_v, half, tok_d, grow_d, sem):
        off = half * HU
        pltpu.async_copy(tok_hbm.at[ids_v.at[pl.ds(off, HU)]], tok_d, sem)
        pltpu.async_copy(geo_hbm.at[gidx_v.at[pl.ds(off, HU)]], grow_d, sem)

    def wait_unit(ids_v, tok_d, grow_d, sem):
        pltpu.make_async_copy(tok_hbm.at[ids_v.at[pl.ds(0, HU)]],
                              tok_d, sem).wait()
        pltpu.make_async_copy(geo_hbm.at[ids_v.at[pl.ds(0, HU)]],
                              grow_d, sem).wait()

    def compute_unit(pb_v, mf_v, co_v, tok_u, grow_u, out_u, half):
        hoff = half * HU

        @plsc.parallel_loop(0, UCHUNK, step=1, unroll=1)
        def chunk_compute(c):
            roff = pl.multiple_of(hoff + c * 16, 16)
            lbase = pl.multiple_of(c * 16, 16)
            pbc = pb_v[pl.ds(roff, 16)]
            mfc = mf_v[pl.ds(roff, 16)]
            coc = co_v[pl.ds(roff, 16)]
            colc = lax.shift_right_logical(coc, 8)
            lanec = jnp.bitwise_and(coc, jnp.int32(255))
            for lane in range(16):
                t = lbase + lane
                pb = pbc[lane]
                colb = colc[lane]
                lo = lanec[lane]
                lov = jnp.full((16,), lo, jnp.int32)
                v16 = grow_u[t, pl.ds(pl.multiple_of(colb, 16), 16)]
                mfv = jnp.take(mfc, jnp.full((16,), lane, jnp.int32), axis=0)
                g0v = jnp.take(v16, lov, axis=0) * mfv
                g1v = jnp.take(v16, lov + 1, axis=0) * mfv
                e = []
                s = None
                q = None
                for j in range(NJ):
                    tv = tok_u[t, pl.ds(j * 16, 16)]
                    pv = posc[pl.ds(pl.multiple_of(pb + j * 16, 16), 16)]
                    ej = tv + pv + (W0[j] * g0v + (W1[j] * g1v + bg[j]))
                    e.append(ej)
                    s = ej if s is None else s + ej
                    q = ej * ej if q is None else q + ej * ej
                meanv = allsum(s) * (1.0 / D)
                varv = jnp.maximum(allsum(q) * (1.0 / D) - meanv * meanv,
                                   jnp.float32(0))
                rstd = _rsqrt16(varv + EPS)
                for j in range(NJ):
                    out_u[t, pl.ds(j * 16, 16)] = \
                        ((e[j] - meanv) * rstd) * gam[j] + bet[j]

    pltpu.sync_copy(ids_hbm.at[pl.ds(row0 * L, L)], idsA.at[pl.ds(0, L)])
    pid_row(idsA, gidxA, pbA, mfA, coA)
    issue_unit(idsA, gidxA, 0, tokH0, growH0, semA)

    def iter_body(rr, carry0):
        def do_row(r, ids_c, gidx_c, pb_c, mf_c, co_c,
                   ids_n, gidx_n, pb_n, mf_n, co_n, first):
            gr = row0 + r
            nxt = jnp.minimum(r + 1, ROWS_PER_W - 1)
            issue_unit(ids_c, gidx_c, 1, tokH1, growH1, semB)
            pltpu.async_copy(ids_hbm.at[pl.ds((row0 + nxt) * L, L)],
                             ids_n.at[pl.ds(0, L)], isem)
            wait_unit(ids_c, tokH0, growH0, semA)
            if first:
                @pl.when(rr > 0)
                def _():
                    pltpu.make_async_copy(
                        outH0, out_hbm.at[pl.ds(0, HU)], osem0).wait()
            else:
                pltpu.make_async_copy(
                    outH0, out_hbm.at[pl.ds(0, HU)], osem0).wait()
            compute_unit(pb_c, mf_c, co_c, tokH0, growH0, outH0, 0)
            pltpu.async_copy(outH0, out_hbm.at[pl.ds(gr * L, HU)], osem0)
            pltpu.make_async_copy(ids_hbm.at[pl.ds(0, L)],
                                  ids_n.at[pl.ds(0, L)], isem).wait()
            pid_row(ids_n, gidx_n, pb_n, mf_n, co_n)
            issue_unit(ids_n, gidx_n, 0, tokH0, growH0, semA)
            wait_unit(ids_c, tokH1, growH1, semB)
            if first:
                @pl.when(rr > 0)
                def _():
                    pltpu.make_async_copy(
                        outH1.at[pl.ds(0, L - HU)],
                        out_hbm.at[pl.ds(0, L - HU)], osem1).wait()
            else:
                pltpu.make_async_copy(
                    outH1.at[pl.ds(0, L - HU)],
                    out_hbm.at[pl.ds(0, L - HU)], osem1).wait()
            compute_unit(pb_c, mf_c, co_c, tokH1, growH1, outH1, 1)
            pltpu.async_copy(outH1.at[pl.ds(0, L - HU)],
                             out_hbm.at[pl.ds(gr * L + HU, L - HU)], osem1)

        do_row(rr * 2, idsA, gidxA, pbA, mfA, coA,
               idsB, gidxB, pbB, mfB, coB, True)
        do_row(rr * 2 + 1, idsB, gidxB, pbB, mfB, coB,
               idsA, gidxA, pbA, mfA, coA, False)
        return carry0

    lax.fori_loop(0, ROWS_PER_W // 2, iter_body, jnp.int32(0))

    wait_unit(idsA, tokH0, growH0, semA)
    pltpu.make_async_copy(outH0, out_hbm.at[pl.ds(0, HU)], osem0).wait()
    pltpu.make_async_copy(outH1.at[pl.ds(0, L - HU)],
                          out_hbm.at[pl.ds(0, L - HU)], osem1).wait()


_sc_embed = pl.kernel(
    _sc_embed_body,
    out_type=jax.ShapeDtypeStruct((B * L, D), jnp.float32),
    mesh=plsc.VectorSubcoreMesh(core_axis_name="c", subcore_axis_name="s"),
    scratch_types=[
        pltpu.VMEM((NPOS * D,), jnp.float32),
        pltpu.VMEM((5 * D,), jnp.float32),
        pltpu.VMEM((LP,), jnp.int32),
        pltpu.VMEM((LP,), jnp.int32),
        pltpu.VMEM((LP,), jnp.int32),
        pltpu.VMEM((LP,), jnp.int32),
        pltpu.VMEM((LP,), jnp.int32),
        pltpu.VMEM((LP,), jnp.int32),
        pltpu.VMEM((LP,), jnp.float32),
        pltpu.VMEM((LP,), jnp.float32),
        pltpu.VMEM((LP,), jnp.int32),
        pltpu.VMEM((LP,), jnp.int32),
        pltpu.VMEM((HU, D), jnp.float32),
        pltpu.VMEM((HU, D), jnp.float32),
        pltpu.VMEM((HU, D), jnp.float32),
        pltpu.VMEM((HU, D), jnp.float32),
        pltpu.VMEM((HU, D), jnp.float32),
        pltpu.VMEM((HU, D), jnp.float32),
        pltpu.SemaphoreType.DMA,
        pltpu.SemaphoreType.DMA,
        pltpu.SemaphoreType.DMA,
        pltpu.SemaphoreType.DMA,
        pltpu.SemaphoreType.DMA,
    ],
)


def kernel(input_ids, geo_dict, token_table, pos_table, W_geo, b_geo,
           ln_gamma, ln_beta):
    ids_flat = input_ids.reshape(-1)
    pos_flat = pos_table.reshape(-1)
    geo_pad = jnp.pad(geo_dict.reshape(-1),
                      (0, GROWS * D - 2 * VOCAB)).reshape(GROWS, D)
    params = jnp.concatenate([W_geo[:, 0], W_geo[:, 1], b_geo, ln_gamma, ln_beta])
    out = _sc_embed(ids_flat, geo_pad, token_table, pos_flat, params)
    return out.reshape(B, L, D)


# speedup vs baseline: 1.9268x; 1.9268x over previous
import jax
import jax.numpy as jnp
from jax import lax
from jax.experimental import pallas as pl
from jax.experimental.pallas import tpu as pltpu
from jax.experimental.pallas import tpu_sc as plsc

VOCAB = 100000
D = 128
PAD_IDX = 0
CLS_IDX = 57255
SEP_IDX = 57256
EPS = 1e-12
B, L = 1024, 200
LP = 208
HALF = 104
NW = 32
ROWS_PER_W = B // NW
NJ = D // 16
NCHUNK = LP // 16
NPOS = 203
GROWS = (2 * VOCAB + D - 1) // D + 1


def _rsqrt16(x):
    i = lax.bitcast_convert_type(x, jnp.int32)
    i = jnp.int32(0x5F3759DF) - lax.shift_right_logical(i, 1)
    y = lax.bitcast_convert_type(i, jnp.float32)
    for _ in range(2):
        y = y * (1.5 - 0.5 * x * y * y)
    return y


def _sc_embed_body(ids_hbm, geo_hbm, tok_hbm, posf_hbm, par_hbm, out_hbm,
                   posc, par_v, idsA, idsB, gidx_v, pb_v, mf_v, co_v,
                   tok_v, grow_v, out_v, semA, isem, osem):
    def allsum(v):
        iota = jnp.arange(16, dtype=jnp.int32)
        for k in (8, 4, 2, 1):
            v = v + jnp.take(v, jnp.bitwise_xor(iota, jnp.int32(k)), axis=0)
        return v

    wid = lax.axis_index("s") * 2 + lax.axis_index("c")
    row0 = wid * ROWS_PER_W

    pltpu.sync_copy(posf_hbm.at[pl.ds(0, 201 * D)], posc.at[pl.ds(0, 201 * D)])
    pltpu.sync_copy(posf_hbm.at[pl.ds(511 * D, D)], posc.at[pl.ds(201 * D, D)])
    pltpu.sync_copy(posf_hbm.at[pl.ds(512 * D, D)], posc.at[pl.ds(202 * D, D)])
    pltpu.sync_copy(par_hbm, par_v)

    z16 = jnp.zeros((16,), jnp.int32)
    for ids_ref in (idsA, idsB):
        ids_ref[pl.ds(192, 16)] = z16

    W0 = [par_v[pl.ds(0 * D + 16 * j, 16)] for j in range(NJ)]
    W1 = [par_v[pl.ds(1 * D + 16 * j, 16)] for j in range(NJ)]
    bg = [par_v[pl.ds(2 * D + 16 * j, 16)] for j in range(NJ)]
    gam = [par_v[pl.ds(3 * D + 16 * j, 16)] for j in range(NJ)]
    bet = [par_v[pl.ds(4 * D + 16 * j, 16)] for j in range(NJ)]

    def do_row(rr, r, ids_v, ids_n, guard_first):
        gr = row0 + r
        nxt = row0 + jnp.minimum(r + 1, ROWS_PER_W - 1)
        pltpu.async_copy(ids_hbm.at[pl.ds(nxt * L, L)],
                         ids_n.at[pl.ds(0, L)], isem)

        for j in range(NCHUNK):
            gidx_v[pl.ds(j * 16, 16)] = lax.shift_right_logical(
                ids_v[pl.ds(j * 16, 16)], 6)

        cps = [
            pltpu.async_copy(tok_hbm.at[ids_v.at[pl.ds(0, HALF)]],
                             tok_v.at[pl.ds(0, HALF)], semA),
            pltpu.async_copy(tok_hbm.at[ids_v.at[pl.ds(HALF, HALF)]],
                             tok_v.at[pl.ds(HALF, HALF)], semA),
            pltpu.async_copy(geo_hbm.at[gidx_v.at[pl.ds(0, HALF)]],
                             grow_v.at[pl.ds(0, HALF)], semA),
            pltpu.async_copy(geo_hbm.at[gidx_v.at[pl.ds(HALF, HALF)]],
                             grow_v.at[pl.ds(HALF, HALF)], semA),
        ]

        carry = jnp.float32(0)
        for j in range(NCHUNK):
            v = ids_v[pl.ds(j * 16, 16)]
            padf = jnp.where(v == PAD_IDX, jnp.float32(1), jnp.float32(0))
            clsf = jnp.where(v == CLS_IDX, jnp.float32(1), jnp.float32(0))
            sepf = jnp.where(v == SEP_IDX, jnp.float32(1), jnp.float32(0))
            mi_f = 1.0 - (padf + clsf + sepf)
            iota = jnp.arange(16, dtype=jnp.int32)
            cs = mi_f
            for k in (1, 2, 4, 8):
                sh = jnp.take(cs, jnp.maximum(iota - k, 0), axis=0)
                cs = cs + jnp.where(iota >= k, sh, jnp.float32(0))
            cs = cs + carry
            pidf = (cs * mi_f + jnp.float32(NPOS - 2) * clsf
                    + jnp.float32(NPOS - 1) * sepf)
            pb_v[pl.ds(j * 16, 16)] = (pidf * D).astype(jnp.int32)
            mf_v[pl.ds(j * 16, 16)] = mi_f
            coff = lax.shift_left(jnp.bitwise_and(v, jnp.int32(63)), 1)
            colb = jnp.bitwise_and(coff, jnp.int32(112))
            co_v[pl.ds(j * 16, 16)] = colb * 256 + (coff - colb)
            carry = cs[15]

        for cp in cps:
            cp.wait()

        if guard_first:
            @pl.when(rr > 0)
            def _():
                pltpu.make_async_copy(out_v.at[pl.ds(0, L)],
                                      out_hbm.at[pl.ds(0, L)], osem).wait()
        else:
            pltpu.make_async_copy(out_v.at[pl.ds(0, L)],
                                  out_hbm.at[pl.ds(0, L)], osem).wait()

        @plsc.parallel_loop(0, NCHUNK, step=1, unroll=1)
        def chunk_compute(c):
            base_t = pl.multiple_of(c * 16, 16)
            pbc = pb_v[pl.ds(base_t, 16)]
            mfc = mf_v[pl.ds(base_t, 16)]
            coc = co_v[pl.ds(base_t, 16)]
            colc = lax.shift_right_logical(coc, 8)
            lanec = jnp.bitwise_and(coc, jnp.int32(255))
            for lane in range(16):
                t = base_t + lane
                pb = pbc[lane]
                colb = colc[lane]
                lo = lanec[lane]
                lov = jnp.full((16,), lo, jnp.int32)
                v16 = grow_v[t, pl.ds(pl.multiple_of(colb, 16), 16)]
                mfv = jnp.take(mfc, jnp.full((16,), lane, jnp.int32), axis=0)
                g0v = jnp.take(v16, lov, axis=0) * mfv
                g1v = jnp.take(v16, lov + 1, axis=0) * mfv
                e = []
                s = None
                q = None
                for j in range(NJ):
                    tv = tok_v[t, pl.ds(j * 16, 16)]
                    pv = posc[pl.ds(pl.multiple_of(pb + j * 16, 16), 16)]
                    ej = tv + pv + (W0[j] * g0v + (W1[j] * g1v + bg[j]))
                    e.append(ej)
                    s = ej if s is None else s + ej
                    q = ej * ej if q is None else q + ej * ej
                meanv = allsum(s) * (1.0 / D)
                varv = jnp.maximum(allsum(q) * (1.0 / D) - meanv * meanv,
                                   jnp.float32(0))
                rstd = _rsqrt16(varv + EPS)
                for j in range(NJ):
                    out_v[t, pl.ds(j * 16, 16)] = \
                        ((e[j] - meanv) * rstd) * gam[j] + bet[j]

        pltpu.async_copy(out_v.at[pl.ds(0, L)],
                         out_hbm.at[pl.ds(gr * L, L)], osem)
        pltpu.make_async_copy(ids_hbm.at[pl.ds(0, L)],
                              ids_n.at[pl.ds(0, L)], isem).wait()

    pltpu.sync_copy(ids_hbm.at[pl.ds(row0 * L, L)], idsA.at[pl.ds(0, L)])

    def iter_body(rr, carry0):
        do_row(rr, rr * 2, idsA, idsB, True)
        do_row(rr, rr * 2 + 1, idsB, idsA, False)
        return carry0

    lax.fori_loop(0, ROWS_PER_W // 2, iter_body, jnp.int32(0))

    pltpu.make_async_copy(out_v.at[pl.ds(0, L)],
                          out_hbm.at[pl.ds(0, L)], osem).wait()


_sc_embed = pl.kernel(
    _sc_embed_body,
    out_type=jax.ShapeDtypeStruct((B * L, D), jnp.float32),
    mesh=plsc.VectorSubcoreMesh(core_axis_name="c", subcore_axis_name="s"),
    scratch_types=[
        pltpu.VMEM((NPOS * D,), jnp.float32),
        pltpu.VMEM((5 * D,), jnp.float32),
        pltpu.VMEM((LP,), jnp.int32),
        pltpu.VMEM((LP,), jnp.int32),
        pltpu.VMEM((LP,), jnp.int32),
        pltpu.VMEM((LP,), jnp.int32),
        pltpu.VMEM((LP,), jnp.float32),
        pltpu.VMEM((LP,), jnp.int32),
        pltpu.VMEM((LP, D), jnp.float32),
        pltpu.VMEM((LP, D), jnp.float32),
        pltpu.VMEM((LP, D), jnp.float32),
        pltpu.SemaphoreType.DMA,
        pltpu.SemaphoreType.DMA,
        pltpu.SemaphoreType.DMA,
    ],
)


def kernel(input_ids, geo_dict, token_table, pos_table, W_geo, b_geo,
           ln_gamma, ln_beta):
    ids_flat = input_ids.reshape(-1)
    pos_flat = pos_table.reshape(-1)
    geo_pad = jnp.pad(geo_dict.reshape(-1),
                      (0, GROWS * D - 2 * VOCAB)).reshape(GROWS, D)
    params = jnp.concatenate([W_geo[:, 0], W_geo[:, 1], b_geo, ln_gamma, ln_beta])
    out = _sc_embed(ids_flat, geo_pad, token_table, pos_flat, params)
    return out.reshape(B, L, D)
